# baseline (speedup 1.0000x reference)
import functools

import jax
import jax.numpy as jnp
from jax import lax
from jax.experimental import pallas as pl
from jax.experimental.pallas import tpu as pltpu
from jax.experimental.pallas import tpu_sc as plsc

D = 256
T = 512
TROWS = 513
P_ROWS = 1024
NC = 2
NS = 16
NW = NC * NS
ROWS_PER_W = T // NW
P_PER_TILE = P_ROWS // NS

_mesh = plsc.VectorSubcoreMesh(core_axis_name="c", subcore_axis_name="s")


@functools.partial(
    pl.kernel,
    out_type=jax.ShapeDtypeStruct((T, T, D), jnp.float32),
    mesh=_mesh,
    compiler_params=pltpu.CompilerParams(use_tc_tiling_on_sc=False),
    scratch_types=[
        pltpu.VMEM_SHARED((P_ROWS, D), jnp.float32),
        pltpu.SemaphoreType.DMA,
    ],
)
def _rpe_sc(table_hbm, out_hbm, p_sh, sem):
    c = lax.axis_index("c")
    s = lax.axis_index("s")
    wid = s * NC + c

    p_base = s * P_PER_TILE
    fills = []
    for m in range(P_PER_TILE):
        p_row = p_base + m
        src_row = jnp.clip(p_row - 256, 0, TROWS - 1)
        fills.append(pltpu.async_copy(
            table_hbm.at[pl.ds(src_row, 1), :],
            p_sh.at[pl.ds(p_row, 1), :],
            sem))
    for cp in fills:
        cp.wait()
    plsc.subcore_barrier()

    i0 = wid * ROWS_PER_W
    outs = []
    for r in range(ROWS_PER_W):
        i = i0 + r
        outs.append(pltpu.async_copy(
            p_sh.at[pl.ds(T - i, T), :],
            out_hbm.at[i],
            sem))
    for cp in outs:
        cp.wait()


def kernel(length, table):
    del length
    return _rpe_sc(table)


# speedup vs baseline: 1.9353x; 1.9353x over previous
import functools

import jax
import jax.numpy as jnp
from jax import lax
from jax.experimental import pallas as pl
from jax.experimental.pallas import tpu as pltpu
from jax.experimental.pallas import tpu_sc as plsc

D = 256
T = 512
TROWS = 513
Q_ROWS = 1024
NC = 2
NS = 16
NPH = 4
CHUNK = Q_ROWS // NS
ROWS_PER_TILE_PER_PH = 4

_mesh = plsc.VectorSubcoreMesh(core_axis_name="c", subcore_axis_name="s")


@functools.partial(
    pl.kernel,
    out_type=jax.ShapeDtypeStruct((T, T, D), jnp.float32),
    mesh=_mesh,
    scratch_types=[
        pltpu.VMEM_SHARED((NPH, Q_ROWS, D), jnp.float32),
        pltpu.VMEM((CHUNK,), jnp.int32),
        pltpu.VMEM((CHUNK, D), jnp.float32),
        pltpu.SemaphoreType.DMA,
    ],
)
def _rpe_sc(table_hbm, out_hbm, q_sh, idx_v, rows_v, sem):
    c = lax.axis_index("c")
    s = lax.axis_index("s")

    lane = lax.iota(jnp.int32, 16)
    for t in range(NPH):
        phi = NPH * c + t
        base = CHUNK * s + phi - 256
        for v in range(CHUNK // 16):
            idx_v[pl.ds(16 * v, 16)] = jnp.clip(
                lane + (base + 16 * v), 0, TROWS - 1)
        pltpu.async_copy(table_hbm.at[idx_v], rows_v, sem).wait()
        pltpu.async_copy(
            rows_v, q_sh.at[t, pl.ds(CHUNK * s, CHUNK), :], sem).wait()
    plsc.subcore_barrier()

    outs = []
    for t in range(NPH):
        phi = NPH * c + t
        rem = (8 - phi) % 8
        off = jnp.where(phi > 0, 1, 0)
        for kk in range(ROWS_PER_TILE_PER_PH):
            k = ROWS_PER_TILE_PER_PH * s + kk
            i = 8 * k + rem
            a = 64 - k - off
            outs.append(pltpu.async_copy(
                q_sh.at[t, pl.ds(8 * a, T), :],
                out_hbm.at[i],
                sem))
    for cp in outs:
        cp.wait()


def kernel(length, table):
    del length
    return _rpe_sc(table)


# speedup vs baseline: 1.9420x; 1.0034x over previous
import functools

import jax
import jax.numpy as jnp
from jax import lax
from jax.experimental import pallas as pl
from jax.experimental.pallas import tpu as pltpu
from jax.experimental.pallas import tpu_sc as plsc

D = 256
T = 512
TROWS = 513
Q_ROWS = 1024
NC = 2
NS = 16
NPH = 4
CHUNK = Q_ROWS // NS
ROWS_PER_TILE_PER_PH = 4

_mesh = plsc.VectorSubcoreMesh(core_axis_name="c", subcore_axis_name="s")


@functools.partial(
    pl.kernel,
    out_type=jax.ShapeDtypeStruct((T, T, D), jnp.float32),
    mesh=_mesh,
    scratch_types=[
        pltpu.VMEM_SHARED((NPH, Q_ROWS, D), jnp.float32),
        pltpu.VMEM((3, CHUNK), jnp.int32),
        pltpu.VMEM((CHUNK, D), jnp.float32),
        pltpu.VMEM((CHUNK, D), jnp.float32),
        pltpu.VMEM((CHUNK, D), jnp.float32),
        pltpu.SemaphoreType.DMA,
        pltpu.SemaphoreType.DMA,
        pltpu.SemaphoreType.DMA,
        pltpu.SemaphoreType.DMA,
        pltpu.SemaphoreType.DMA,
        pltpu.SemaphoreType.DMA,
        pltpu.SemaphoreType.DMA,
    ],
)
def _rpe_sc(table_hbm, out_hbm, q_sh, idx_v, ra, rb, rc,
            ga, gb, gc, pa, pb, pc, sem):
    c = lax.axis_index("c")
    s = lax.axis_index("s")
    rows = [ra, rb, rc]
    gsems = [ga, gb, gc]
    psems = [pa, pb, pc]

    lane = lax.iota(jnp.int32, 16)

    def set_idx(t, b):
        phi = NPH * c + t
        base = CHUNK * s + phi - 256
        for v in range(CHUNK // 16):
            idx_v[b, pl.ds(16 * v, 16)] = jnp.clip(
                lane + (base + 16 * v), 0, TROWS - 1)

    def gather(t, b):
        return pltpu.async_copy(table_hbm.at[idx_v.at[b]], rows[b], gsems[b])

    def publish(t, b):
        return pltpu.async_copy(
            rows[b], q_sh.at[t, pl.ds(CHUNK * s, CHUNK), :], psems[b])

    set_idx(0, 0)
    g0 = gather(0, 0)
    set_idx(1, 1)
    g1 = gather(1, 1)
    set_idx(2, 2)
    g2 = gather(2, 2)
    g0.wait()
    p0 = publish(0, 0)
    g1.wait()
    p1 = publish(1, 1)
    g2.wait()
    p2 = publish(2, 2)
    p0.wait()
    set_idx(3, 0)
    g3 = gather(3, 0)
    g3.wait()
    p3 = publish(3, 0)
    p1.wait()
    p2.wait()
    p3.wait()
    plsc.subcore_barrier()

    outs = []
    for t in range(NPH):
        phi = NPH * c + t
        rem = (8 - phi) % 8
        off = jnp.where(phi > 0, 1, 0)
        for kk in range(ROWS_PER_TILE_PER_PH):
            k = ROWS_PER_TILE_PER_PH * s + kk
            i = 8 * k + rem
            a = 64 - k - off
            outs.append(pltpu.async_copy(
                q_sh.at[t, pl.ds(8 * a, T), :],
                out_hbm.at[i],
                sem))
    for cp in outs:
        cp.wait()


def kernel(length, table):
    del length
    return _rpe_sc(table)


# speedup vs baseline: 2.0091x; 1.0346x over previous
import functools

import jax
import jax.numpy as jnp
from jax import lax
from jax.experimental import pallas as pl
from jax.experimental.pallas import tpu as pltpu
from jax.experimental.pallas import tpu_sc as plsc

D = 256
T = 512
TROWS = 513
Q_ROWS = 1024
NC = 2
NS = 16
NPH = 4
CHUNK = Q_ROWS // NS
ROWS_PER_TILE_PER_PH = 4

_mesh = plsc.VectorSubcoreMesh(core_axis_name="c", subcore_axis_name="s")


@functools.partial(
    pl.kernel,
    out_type=jax.ShapeDtypeStruct((T, T, D), jnp.float32),
    mesh=_mesh,
    scratch_types=[
        pltpu.VMEM_SHARED((NPH, Q_ROWS, D), jnp.float32),
        pltpu.VMEM((CHUNK,), jnp.int32),
        pltpu.VMEM((CHUNK,), jnp.int32),
        pltpu.VMEM((CHUNK,), jnp.int32),
        pltpu.VMEM((CHUNK, D), jnp.float32),
        pltpu.VMEM((CHUNK, D), jnp.float32),
        pltpu.VMEM((CHUNK, D), jnp.float32),
        pltpu.SemaphoreType.DMA,
        pltpu.SemaphoreType.DMA,
        pltpu.SemaphoreType.DMA,
        pltpu.SemaphoreType.DMA,
        pltpu.SemaphoreType.DMA,
        pltpu.SemaphoreType.DMA,
        pltpu.SemaphoreType.DMA,
    ],
)
def _rpe_sc(table_hbm, out_hbm, q_sh, ia, ib, ic, ra, rb, rc,
            ga, gb, gc, pa, pb, pc, sem):
    c = lax.axis_index("c")
    s = lax.axis_index("s")
    idxs = [ia, ib, ic]
    rows = [ra, rb, rc]
    gsems = [ga, gb, gc]
    psems = [pa, pb, pc]

    lane = lax.iota(jnp.int32, 16)

    def set_idx(t, b):
        phi = NPH * c + t
        base = CHUNK * s + phi - 256
        for v in range(CHUNK // 16):
            idxs[b][pl.ds(16 * v, 16)] = jnp.clip(
                lane + (base + 16 * v), 0, TROWS - 1)

    def gather(t, b):
        return pltpu.async_copy(table_hbm.at[idxs[b]], rows[b], gsems[b])

    def publish(t, b):
        return pltpu.async_copy(
            rows[b], q_sh.at[t, pl.ds(CHUNK * s, CHUNK), :], psems[b])

    outs = []

    def fire(t):
        phi = NPH * c + t
        rem = (8 - phi) % 8
        off = jnp.where(phi > 0, 1, 0)
        for kk in range(ROWS_PER_TILE_PER_PH):
            k = ROWS_PER_TILE_PER_PH * s + kk
            i = 8 * k + rem
            a = 64 - k - off
            outs.append(pltpu.async_copy(
                q_sh.at[t, pl.ds(8 * a, T), :],
                out_hbm.at[i],
                sem))

    set_idx(0, 0)
    g0 = gather(0, 0)
    set_idx(1, 1)
    g1 = gather(1, 1)
    set_idx(2, 2)
    g2 = gather(2, 2)
    g0.wait()
    p0 = publish(0, 0)
    p0.wait()
    set_idx(3, 0)
    g3 = gather(3, 0)
    plsc.subcore_barrier()
    fire(0)
    g1.wait()
    p1 = publish(1, 1)
    p1.wait()
    plsc.subcore_barrier()
    fire(1)
    g2.wait()
    p2 = publish(2, 2)
    p2.wait()
    plsc.subcore_barrier()
    fire(2)
    g3.wait()
    p3 = publish(3, 0)
    p3.wait()
    plsc.subcore_barrier()
    fire(3)
    for cp in outs:
        cp.wait()


def kernel(length, table):
    del length
    return _rpe_sc(table)


# speedup vs baseline: 2.5285x; 1.2585x over previous
import functools

import jax
import jax.numpy as jnp
from jax import lax
from jax.experimental import pallas as pl
from jax.experimental.pallas import tpu as pltpu
from jax.experimental.pallas import tpu_sc as plsc

D = 256
T = 512
TROWS = 513
Q_ROWS = 1024
NC = 2
NS = 16
NPH = 4
CHUNK = Q_ROWS // NS
ROWS_PER_TILE_PER_PH = 4
BLK = 72
LANES = 16

_mesh = plsc.VectorSubcoreMesh(core_axis_name="c", subcore_axis_name="s")


@functools.partial(
    pl.kernel,
    out_type=jax.ShapeDtypeStruct((T, T, D), jnp.float32),
    mesh=_mesh,
    scratch_types=[
        pltpu.VMEM_SHARED((NPH, Q_ROWS, D), jnp.float32),
        pltpu.VMEM((BLK, D), jnp.float32),
        pltpu.VMEM((LANES,), jnp.int32),
        pltpu.VMEM((LANES, D), jnp.float32),
        pltpu.VMEM((CHUNK, D), jnp.float32),
        pltpu.VMEM((CHUNK, D), jnp.float32),
        pltpu.SemaphoreType.DMA,
        pltpu.SemaphoreType.DMA,
        pltpu.SemaphoreType.DMA,
        pltpu.SemaphoreType.DMA,
    ],
)
def _rpe_sc(table_hbm, out_hbm, q_sh, blk_v, idx_l, last_v, ra, rb,
            lsem, pa, pb, sem):
    c = lax.axis_index("c")
    s = lax.axis_index("s")
    rows = [ra, rb]
    psems = [pa, pb]

    base = pl.multiple_of(jnp.clip(CHUNK * s - 256, 0, 440), 8)
    lane = lax.iota(jnp.int32, LANES)
    idx_l[pl.ds(0, LANES)] = lane * 0 + (TROWS - 1)
    ld_b = pltpu.async_copy(table_hbm.at[pl.ds(base, BLK), :], blk_v, lsem)
    ld_b.wait()
    ld_l = pltpu.async_copy(table_hbm.at[idx_l], last_v, lsem)
    ld_l.wait()

    last_regs = [last_v[0, pl.ds(LANES * u, LANES)] for u in range(D // LANES)]

    def build(t, b):
        phi = NPH * c + t
        u0 = CHUNK * s + phi - 256

        def body(r, carry):
            v = u0 + r
            vloc = jnp.clip(v, 0, TROWS - 1) - base
            use_last = vloc >= BLK
            vl = jnp.minimum(vloc, BLK - 1)
            for u in range(D // LANES):
                x = blk_v[vl, pl.ds(LANES * u, LANES)]
                rows[b][r, pl.ds(LANES * u, LANES)] = jnp.where(
                    use_last, last_regs[u], x)
            return carry

        lax.fori_loop(0, CHUNK, body, 0)

    def publish(t, b):
        return pltpu.async_copy(
            rows[b], q_sh.at[t, pl.ds(CHUNK * s, CHUNK), :], psems[b])

    outs = []

    def fire(t):
        phi = NPH * c + t
        rem = (8 - phi) % 8
        off = jnp.where(phi > 0, 1, 0)
        for kk in range(ROWS_PER_TILE_PER_PH):
            k = ROWS_PER_TILE_PER_PH * s + kk
            i = 8 * k + rem
            a = 64 - k - off
            outs.append(pltpu.async_copy(
                q_sh.at[t, pl.ds(8 * a, T), :],
                out_hbm.at[i],
                sem))

    for t in range(NPH):
        b = t % 2
        build(t, b)
        pub = publish(t, b)
        pub.wait()
        plsc.subcore_barrier()
        fire(t)
    for cp in outs:
        cp.wait()


def kernel(length, table):
    del length
    return _rpe_sc(table)
